# Initial kernel scaffold; baseline (speedup 1.0000x reference)
import functools

import jax
import jax.numpy as jnp
from jax import lax
from jax.experimental import pallas as pl
from jax.experimental.pallas import tpu as pltpu
from jax.experimental.pallas import tpu_sc as plsc

_N_NODES = 10000
_N_EDGES = 320000
_D = 128
_N_PRED = 200000

_NC = 2
_NS = 16
_NW = _NC * _NS
_L = 16

_CHUNK = 128
_EDGE_CHUNKS = _N_EDGES // _CHUNK
_PRED_PAD = -(-_N_PRED // _CHUNK) * _CHUNK
_PRED_CHUNKS = _PRED_PAD // _CHUNK

_ROWS_PER_TILE = _N_NODES // _NS
_ZROWS = 125

_mesh = plsc.VectorSubcoreMesh(core_axis_name="c", subcore_axis_name="s")


@functools.partial(
    pl.kernel,
    out_type=jax.ShapeDtypeStruct((_NC, _N_NODES, _D), jnp.float32),
    mesh=_mesh,
    scratch_types=[
        pltpu.VMEM((_CHUNK,), jnp.int32),
        pltpu.VMEM((_CHUNK,), jnp.int32),
        pltpu.VMEM((_CHUNK,), jnp.float32),
        pltpu.VMEM((_CHUNK, _D), jnp.float32),
        pltpu.VMEM((_ZROWS, _D), jnp.float32),
        pltpu.VMEM_SHARED((_N_NODES, _D), jnp.float32),
        pltpu.SemaphoreType.DMA,
    ],
)
def _stage_a(x_hbm, src_hbm, dst_hbm, w_hbm, out_hbm,
             idx_s, idx_d, w_v, rows, zbuf, agg_sh, sem):
    c = lax.axis_index("c")
    s = lax.axis_index("s")
    wid = s * _NC + c

    zv = jnp.zeros((_L,), jnp.float32)

    def _zrow(i, _):
        for j in range(_D // _L):
            zbuf[i, pl.ds(j * _L, _L)] = zv
        return 0

    lax.fori_loop(0, _ZROWS, _zrow, 0)
    for t in range(_ROWS_PER_TILE // _ZROWS):
        pltpu.sync_copy(zbuf, agg_sh.at[pl.ds(s * _ROWS_PER_TILE + t * _ZROWS, _ZROWS)])
    plsc.subcore_barrier()

    n_iter = -(-_EDGE_CHUNKS // _NW)

    def _chunk(k, _):
        g = wid + k * _NW

        @pl.when(g < _EDGE_CHUNKS)
        def _():
            pltpu.sync_copy(src_hbm.at[g], idx_s)
            pltpu.sync_copy(dst_hbm.at[g], idx_d)
            pltpu.sync_copy(w_hbm.at[g], w_v)
            pltpu.async_copy(x_hbm.at[idx_s], rows, sem).wait()

            def _scale(r, _):
                wb = plsc.load_gather(w_v, [jnp.broadcast_to(r, (_L,))])
                for j in range(_D // _L):
                    sl = pl.ds(j * _L, _L)
                    rows[r, sl] = rows[r, sl] * wb
                return 0

            lax.fori_loop(0, _CHUNK, _scale, 0)
            pltpu.sync_copy(rows, agg_sh.at[idx_d], add=True)

        return 0

    lax.fori_loop(0, n_iter, _chunk, 0)
    plsc.subcore_barrier()

    for t in range(_ROWS_PER_TILE // _ZROWS):
        sl = pl.ds(s * _ROWS_PER_TILE + t * _ZROWS, _ZROWS)
        pltpu.sync_copy(agg_sh.at[sl], out_hbm.at[c, sl])


def _mm_body(p_ref, w_ref, o_ref):
    o_ref[...] = jnp.dot(p_ref[0] + p_ref[1], w_ref[...],
                         preferred_element_type=jnp.float32)


def _matmul(partials, W):
    blk = 400
    return pl.pallas_call(
        _mm_body,
        grid=(_N_NODES // blk,),
        in_specs=[
            pl.BlockSpec((_NC, blk, _D), lambda i: (0, i, 0)),
            pl.BlockSpec((_D, _D), lambda i: (0, 0)),
        ],
        out_specs=pl.BlockSpec((blk, _D), lambda i: (i, 0)),
        out_shape=jax.ShapeDtypeStruct((_N_NODES, _D), jnp.float32),
    )(partials, W)


@functools.partial(
    pl.kernel,
    out_type=jax.ShapeDtypeStruct((_PRED_CHUNKS, _CHUNK), jnp.float32),
    mesh=_mesh,
    scratch_types=[
        pltpu.VMEM((_CHUNK,), jnp.int32),
        pltpu.VMEM((_CHUNK,), jnp.int32),
        pltpu.VMEM((_CHUNK, _D), jnp.float32),
        pltpu.VMEM((_CHUNK, _D), jnp.float32),
        pltpu.VMEM((_CHUNK,), jnp.float32),
        pltpu.SemaphoreType.DMA,
    ],
)
def _stage_c(h_hbm, e0_hbm, e1_hbm, out_hbm, i0, i1, ra, rb, dots, sem):
    c = lax.axis_index("c")
    s = lax.axis_index("s")
    wid = s * _NC + c
    n_iter = -(-_PRED_CHUNKS // _NW)
    lane = lax.iota(jnp.int32, _L)

    def _chunk(k, _):
        g = wid + k * _NW

        @pl.when(g < _PRED_CHUNKS)
        def _():
            pltpu.sync_copy(e0_hbm.at[g], i0)
            pltpu.sync_copy(e1_hbm.at[g], i1)
            pltpu.async_copy(h_hbm.at[i0], ra, sem).wait()
            pltpu.async_copy(h_hbm.at[i1], rb, sem).wait()

            def _group(grp, _):
                rows16 = grp * _L + lane

                def _col(cc, acc):
                    ci = jnp.broadcast_to(cc, (_L,))
                    va = plsc.load_gather(ra, [rows16, ci])
                    vb = plsc.load_gather(rb, [rows16, ci])
                    return acc + va * vb

                acc = lax.fori_loop(0, _D, _col, jnp.zeros((_L,), jnp.float32))
                dots[pl.ds(grp * _L, _L)] = acc
                return 0

            lax.fori_loop(0, _CHUNK // _L, _group, 0)
            pltpu.sync_copy(dots, out_hbm.at[g])

        return 0

    lax.fori_loop(0, n_iter, _chunk, 0)


def kernel(x, edge_index, edge_weight, edges, W):
    src = edge_index[0].astype(jnp.int32).reshape(_EDGE_CHUNKS, _CHUNK)
    dst = edge_index[1].astype(jnp.int32).reshape(_EDGE_CHUNKS, _CHUNK)
    ew = edge_weight.reshape(_EDGE_CHUNKS, _CHUNK)
    pad = _PRED_PAD - _N_PRED
    e0 = jnp.concatenate([edges[0].astype(jnp.int32),
                          jnp.zeros((pad,), jnp.int32)]).reshape(_PRED_CHUNKS, _CHUNK)
    e1 = jnp.concatenate([edges[1].astype(jnp.int32),
                          jnp.zeros((pad,), jnp.int32)]).reshape(_PRED_CHUNKS, _CHUNK)

    partials = _stage_a(x, src, dst, ew)
    h = _matmul(partials, W)
    dots = _stage_c(h, e0, e1)
    return dots.reshape(-1)[:_N_PRED]


# speedup vs baseline: 2.0239x; 2.0239x over previous
import functools

import jax
import jax.numpy as jnp
from jax import lax
from jax.experimental import pallas as pl
from jax.experimental.pallas import tpu as pltpu
from jax.experimental.pallas import tpu_sc as plsc

_N_NODES = 10000
_N_EDGES = 320000
_D = 128
_N_PRED = 200000

_NC = 2
_NS = 16
_NW = _NC * _NS
_L = 16

_CHUNK = 128
_EDGE_CHUNKS = _N_EDGES // _CHUNK
_PRED_PAD = -(-_N_PRED // _CHUNK) * _CHUNK
_PRED_CHUNKS = _PRED_PAD // _CHUNK

_N_PAD = 10240
_ROWS_PER_TILE = _N_PAD // _NS
_ZROWS = 128

_mesh = plsc.VectorSubcoreMesh(core_axis_name="c", subcore_axis_name="s")
_sc_params = pltpu.CompilerParams(needs_layout_passes=False)


@functools.partial(
    pl.kernel,
    out_type=jax.ShapeDtypeStruct((_NC, _N_PAD, _D), jnp.float32),
    mesh=_mesh,
    scratch_types=[
        pltpu.VMEM((_CHUNK,), jnp.int32),
        pltpu.VMEM((_CHUNK,), jnp.int32),
        pltpu.VMEM((_CHUNK,), jnp.float32),
        pltpu.VMEM((_CHUNK, _D), jnp.float32),
        pltpu.VMEM((_ZROWS, _D), jnp.float32),
        pltpu.VMEM_SHARED((_N_PAD, _D), jnp.float32),
        pltpu.SemaphoreType.DMA,
    ],
    compiler_params=_sc_params,
)
def _stage_a(x_hbm, src_hbm, dst_hbm, w_hbm, out_hbm,
             idx_s, idx_d, w_v, rows, zbuf, agg_sh, sem):
    c = lax.axis_index("c")
    s = lax.axis_index("s")
    wid = s * _NC + c

    zv = jnp.zeros((_L,), jnp.float32)

    def _zrow(i, _):
        for j in range(_D // _L):
            zbuf[i, pl.ds(j * _L, _L)] = zv
        return 0

    lax.fori_loop(0, _ZROWS, _zrow, 0)
    for t in range(_ROWS_PER_TILE // _ZROWS):
        pltpu.sync_copy(zbuf, agg_sh.at[pl.ds(s * _ROWS_PER_TILE + t * _ZROWS, _ZROWS)])
    plsc.subcore_barrier()

    n_iter = -(-_EDGE_CHUNKS // _NW)

    def _chunk(k, _):
        g = wid + k * _NW

        @pl.when(g < _EDGE_CHUNKS)
        def _():
            pltpu.sync_copy(src_hbm.at[g], idx_s)
            pltpu.sync_copy(dst_hbm.at[g], idx_d)
            pltpu.sync_copy(w_hbm.at[g], w_v)
            pltpu.async_copy(x_hbm.at[idx_s], rows, sem).wait()

            def _scale(r, _):
                wb = plsc.load_gather(w_v, [jnp.broadcast_to(r, (_L,))])
                for j in range(_D // _L):
                    sl = pl.ds(j * _L, _L)
                    rows[r, sl] = rows[r, sl] * wb
                return 0

            lax.fori_loop(0, _CHUNK, _scale, 0)
            pltpu.sync_copy(rows, agg_sh.at[idx_d], add=True)

        return 0

    lax.fori_loop(0, n_iter, _chunk, 0)
    plsc.subcore_barrier()

    for t in range(_ROWS_PER_TILE // _ZROWS):
        sl = pl.ds(s * _ROWS_PER_TILE + t * _ZROWS, _ZROWS)
        pltpu.sync_copy(agg_sh.at[sl], out_hbm.at[c, sl])


def _mm_body(p_ref, w_ref, o_ref):
    o_ref[...] = jnp.dot(p_ref[0] + p_ref[1], w_ref[...],
                         preferred_element_type=jnp.float32)


def _matmul(partials, W):
    blk = 512
    return pl.pallas_call(
        _mm_body,
        grid=(_N_PAD // blk,),
        in_specs=[
            pl.BlockSpec((_NC, blk, _D), lambda i: (0, i, 0)),
            pl.BlockSpec((_D, _D), lambda i: (0, 0)),
        ],
        out_specs=pl.BlockSpec((blk, _D), lambda i: (i, 0)),
        out_shape=jax.ShapeDtypeStruct((_N_PAD, _D), jnp.float32),
    )(partials, W)


@functools.partial(
    pl.kernel,
    out_type=jax.ShapeDtypeStruct((_PRED_CHUNKS, _CHUNK), jnp.float32),
    mesh=_mesh,
    scratch_types=[
        pltpu.VMEM((_CHUNK,), jnp.int32),
        pltpu.VMEM((_CHUNK,), jnp.int32),
        pltpu.VMEM((_CHUNK, _D), jnp.float32),
        pltpu.VMEM((_CHUNK, _D), jnp.float32),
        pltpu.VMEM((_CHUNK,), jnp.float32),
        pltpu.SemaphoreType.DMA,
    ],
    compiler_params=_sc_params,
)
def _stage_c(h_hbm, e0_hbm, e1_hbm, out_hbm, i0, i1, ra, rb, dots, sem):
    c = lax.axis_index("c")
    s = lax.axis_index("s")
    wid = s * _NC + c
    n_iter = -(-_PRED_CHUNKS // _NW)
    lane = lax.iota(jnp.int32, _L)

    def _chunk(k, _):
        g = wid + k * _NW

        @pl.when(g < _PRED_CHUNKS)
        def _():
            pltpu.sync_copy(e0_hbm.at[g], i0)
            pltpu.sync_copy(e1_hbm.at[g], i1)
            pltpu.async_copy(h_hbm.at[i0], ra, sem).wait()
            pltpu.async_copy(h_hbm.at[i1], rb, sem).wait()

            def _group(grp, _):
                rows16 = grp * _L + lane

                def _col(cc, acc):
                    ci = jnp.broadcast_to(cc, (_L,))
                    va = plsc.load_gather(ra, [rows16, ci])
                    vb = plsc.load_gather(rb, [rows16, ci])
                    return acc + va * vb

                acc = lax.fori_loop(0, _D, _col, jnp.zeros((_L,), jnp.float32))
                dots[pl.ds(grp * _L, _L)] = acc
                return 0

            lax.fori_loop(0, _CHUNK // _L, _group, 0)
            pltpu.sync_copy(dots, out_hbm.at[g])

        return 0

    lax.fori_loop(0, n_iter, _chunk, 0)


def kernel(x, edge_index, edge_weight, edges, W):
    src = edge_index[0].astype(jnp.int32).reshape(_EDGE_CHUNKS, _CHUNK)
    dst = edge_index[1].astype(jnp.int32).reshape(_EDGE_CHUNKS, _CHUNK)
    ew = edge_weight.reshape(_EDGE_CHUNKS, _CHUNK)
    pad = _PRED_PAD - _N_PRED
    e0 = jnp.concatenate([edges[0].astype(jnp.int32),
                          jnp.zeros((pad,), jnp.int32)]).reshape(_PRED_CHUNKS, _CHUNK)
    e1 = jnp.concatenate([edges[1].astype(jnp.int32),
                          jnp.zeros((pad,), jnp.int32)]).reshape(_PRED_CHUNKS, _CHUNK)

    partials = _stage_a(x, src, dst, ew)
    h = _matmul(partials, W)
    dots = _stage_c(h, e0, e1)
    return dots.reshape(-1)[:_N_PRED]


# speedup vs baseline: 2.6010x; 1.2851x over previous
import functools

import jax
import jax.numpy as jnp
from jax import lax
from jax.experimental import pallas as pl
from jax.experimental.pallas import tpu as pltpu
from jax.experimental.pallas import tpu_sc as plsc

_N_NODES = 10000
_N_EDGES = 320000
_D = 128
_N_PRED = 200000

_NC = 2
_NS = 16
_NW = _NC * _NS
_L = 16

_CHUNK = 128
_PC_W = 49
_P_PAD = _NW * _PC_W * _CHUNK

_ECHUNK = 128
_EC_W = 79
_E_PAD = _NW * _EC_W * _ECHUNK
_NSLOT = 6
_NBUF_A = 2

_N_PAD = 10240
_ROWS_PER_TILE = _N_PAD // _NS

_mesh = plsc.VectorSubcoreMesh(core_axis_name="c", subcore_axis_name="s")
_sc_params = pltpu.CompilerParams(needs_layout_passes=False)


@functools.partial(
    pl.kernel,
    out_type=jax.ShapeDtypeStruct((_NC, _N_PAD, _D), jnp.float32),
    mesh=_mesh,
    scratch_types=[
        [pltpu.VMEM((_ECHUNK,), jnp.int32)] * _NBUF_A,
        [pltpu.VMEM((_ECHUNK,), jnp.int32)] * _NBUF_A,
        [pltpu.VMEM((_ECHUNK,), jnp.float32)] * _NBUF_A,
        pltpu.VMEM((_NBUF_A, _ECHUNK, _D), jnp.float32),
        pltpu.VMEM_SHARED((_N_PAD, _D), jnp.float32),
        [pltpu.SemaphoreType.DMA] * _NBUF_A,
        [pltpu.SemaphoreType.DMA] * _NBUF_A,
        [pltpu.SemaphoreType.DMA] * _NBUF_A,
    ],
    compiler_params=_sc_params,
)
def _stage_a(x_hbm, src_hbm, dst_hbm, w_hbm, zeros_hbm, out_hbm,
             src_b, dst_b, w_b, rows, agg_sh, isem, gsem, ssem):
    c = lax.axis_index("c")
    s = lax.axis_index("s")
    wid = s * _NC + c

    def _issue_idx(j, b):
        pltpu.async_copy(src_hbm.at[wid, j], src_b[b], isem[b])
        pltpu.async_copy(dst_hbm.at[wid, j], dst_b[b], isem[b])
        pltpu.async_copy(w_hbm.at[wid, j], w_b[b], isem[b])

    def _wait_idx(b):
        pltpu.make_async_copy(src_hbm.at[0, 0], src_b[b], isem[b]).wait()
        pltpu.make_async_copy(dst_hbm.at[0, 0], dst_b[b], isem[b]).wait()
        pltpu.make_async_copy(w_hbm.at[0, 0], w_b[b], isem[b]).wait()

    def _issue_gather(b):
        pltpu.async_copy(x_hbm.at[src_b[b]], rows.at[b], gsem[b])

    def _wait_gather(b):
        pltpu.make_async_copy(x_hbm.at[pl.ds(0, _ECHUNK)], rows.at[b],
                              gsem[b]).wait()

    def _wait_scatter(b):
        pltpu.make_async_copy(rows.at[b], agg_sh.at[pl.ds(0, _ECHUNK)],
                              ssem[b]).wait()

    _issue_idx(0, 0)
    pltpu.sync_copy(zeros_hbm, agg_sh.at[pl.ds(s * _ROWS_PER_TILE, _ROWS_PER_TILE)])
    plsc.subcore_barrier()

    _wait_idx(0)
    _issue_gather(0)

    def _body(g, _):
        for b in range(_NBUF_A):
            j = g * _NBUF_A + b

            @pl.when(j < _EC_W)
            def _():
                @pl.when(j >= 1)
                def _():
                    _wait_scatter(1 - b)

                @pl.when(j + 1 < _EC_W)
                def _():
                    _issue_idx(j + 1, 1 - b)

                _wait_gather(b)

                def _row(i, _):
                    for u in range(2):
                        r = i * 2 + u
                        wb = plsc.load_gather(w_b[b], [jnp.broadcast_to(r, (_L,))])
                        for t in range(_D // _L):
                            sl = pl.ds(t * _L, _L)
                            rows[b, r, sl] = rows[b, r, sl] * wb
                    return 0

                lax.fori_loop(0, _ECHUNK // 2, _row, 0)

                pltpu.async_copy(rows.at[b], agg_sh.at[dst_b[b]], ssem[b],
                                 add=True)

                @pl.when(j + 1 < _EC_W)
                def _():
                    _wait_idx(1 - b)
                    _issue_gather(1 - b)

        return 0

    lax.fori_loop(0, -(-_EC_W // _NBUF_A), _body, 0)
    _wait_scatter((_EC_W - 1) % _NBUF_A)
    plsc.subcore_barrier()

    sl = pl.ds(s * _ROWS_PER_TILE, _ROWS_PER_TILE)
    pltpu.sync_copy(agg_sh.at[sl], out_hbm.at[c, sl])


def _mm_body(p_ref, w_ref, o_ref):
    o_ref[...] = jnp.dot(p_ref[0] + p_ref[1], w_ref[...],
                         preferred_element_type=jnp.float32)


def _matmul(partials, W):
    blk = 512
    return pl.pallas_call(
        _mm_body,
        grid=(_N_PAD // blk,),
        in_specs=[
            pl.BlockSpec((_NC, blk, _D), lambda i: (0, i, 0)),
            pl.BlockSpec((_D, _D), lambda i: (0, 0)),
        ],
        out_specs=pl.BlockSpec((blk, _D), lambda i: (i, 0)),
        out_shape=jax.ShapeDtypeStruct((_N_PAD, _D), jnp.float32),
    )(partials, W)


@functools.partial(
    pl.kernel,
    out_type=jax.ShapeDtypeStruct((_NW, _PC_W, _CHUNK), jnp.float32),
    mesh=_mesh,
    scratch_types=[
        pltpu.VMEM((_PC_W, _CHUNK), jnp.int32),
        pltpu.VMEM((_PC_W, _CHUNK), jnp.int32),
        pltpu.VMEM((2, _CHUNK, _D), jnp.float32),
        pltpu.VMEM((2, _CHUNK, _D), jnp.float32),
        pltpu.VMEM((_PC_W, _CHUNK), jnp.float32),
        [pltpu.SemaphoreType.DMA] * 2,
        [pltpu.SemaphoreType.DMA] * 2,
    ],
    compiler_params=_sc_params,
)
def _stage_c(h_hbm, e0_hbm, e1_hbm, out_hbm, i0, i1, ra, rb, dots, sema, semb):
    c = lax.axis_index("c")
    s = lax.axis_index("s")
    wid = s * _NC + c
    lane = lax.iota(jnp.int32, _L)

    pltpu.sync_copy(e0_hbm.at[wid], i0)
    pltpu.sync_copy(e1_hbm.at[wid], i1)

    for b in range(2):
        pltpu.async_copy(h_hbm.at[i0.at[b]], ra.at[b], sema[b])
        pltpu.async_copy(h_hbm.at[i1.at[b]], rb.at[b], semb[b])

    def _chunk(j, _):
        for b in range(2):

            @pl.when((j * 2 + b) < _PC_W)
            def _():
                jj = j * 2 + b
                pltpu.make_async_copy(h_hbm.at[pl.ds(0, _CHUNK)], ra.at[b],
                                      sema[b]).wait()
                pltpu.make_async_copy(h_hbm.at[pl.ds(0, _CHUNK)], rb.at[b],
                                      semb[b]).wait()

                def _group(grp, _):
                    rows16 = grp * _L + lane
                    z = jnp.zeros((_L,), jnp.float32)

                    def _col(ci, accs):
                        a0, a1, a2, a3 = accs
                        cc = ci * 4
                        acc = []
                        for t, at in enumerate((a0, a1, a2, a3)):
                            civ = jnp.broadcast_to(cc + t, (_L,))
                            va = plsc.load_gather(ra.at[b], [rows16, civ])
                            vb = plsc.load_gather(rb.at[b], [rows16, civ])
                            acc.append(at + va * vb)
                        return tuple(acc)

                    a0, a1, a2, a3 = lax.fori_loop(0, _D // 4, _col, (z, z, z, z))
                    dots[jj, pl.ds(grp * _L, _L)] = (a0 + a1) + (a2 + a3)
                    return 0

                lax.fori_loop(0, _CHUNK // _L, _group, 0)

                @pl.when(jj + 2 < _PC_W)
                def _():
                    pltpu.async_copy(h_hbm.at[i0.at[jj + 2]], ra.at[b], sema[b])
                    pltpu.async_copy(h_hbm.at[i1.at[jj + 2]], rb.at[b], semb[b])

        return 0

    lax.fori_loop(0, -(-_PC_W // 2), _chunk, 0)
    pltpu.sync_copy(dots, out_hbm.at[wid])


def kernel(x, edge_index, edge_weight, edges, W):
    epad = _E_PAD - _N_EDGES
    src = jnp.concatenate([edge_index[0].astype(jnp.int32),
                           jnp.zeros((epad,), jnp.int32)]).reshape(_NW, _EC_W, _ECHUNK)
    dst = jnp.concatenate([edge_index[1].astype(jnp.int32),
                           jnp.zeros((epad,), jnp.int32)]).reshape(_NW, _EC_W, _ECHUNK)
    ew = jnp.concatenate([edge_weight,
                          jnp.zeros((epad,), jnp.float32)]).reshape(_NW, _EC_W, _ECHUNK)
    ppad = _P_PAD - _N_PRED
    e0 = jnp.concatenate([edges[0].astype(jnp.int32),
                          jnp.zeros((ppad,), jnp.int32)]).reshape(_NW, _PC_W, _CHUNK)
    e1 = jnp.concatenate([edges[1].astype(jnp.int32),
                          jnp.zeros((ppad,), jnp.int32)]).reshape(_NW, _PC_W, _CHUNK)
    zeros_blk = jnp.zeros((_ROWS_PER_TILE, _D), jnp.float32)

    partials = _stage_a(x, src, dst, ew, zeros_blk)
    h = _matmul(partials, W)
    dots = _stage_c(h, e0, e1)
    return dots.reshape(-1)[:_N_PRED]
